# Initial kernel scaffold; baseline (speedup 1.0000x reference)
import jax
import jax.numpy as jnp
from jax.experimental import pallas as pl


def kernel(x, W, b):
    raise NotImplementedError("write your pallas kernel here")


# speedup vs baseline: 5.7456x; 5.7456x over previous
import functools

import jax
import jax.numpy as jnp
from jax.experimental import pallas as pl

D_MODEL = 4096
N_EXP = 4
TOK_TILE = 512


def _moe_body(x_ref, wt_ref, b_ref, g_ref, o_ref):
    xb = x_ref[...]
    logits = jnp.dot(xb, wt_ref[...], preferred_element_type=jnp.float32)
    z = (logits + b_ref[...]) + g_ref[...]
    idx = jnp.argmax(z, axis=-1)

    s = jnp.sum(xb, axis=-1)
    nz = jnp.sum((xb != 0.0).astype(jnp.float32), axis=-1)

    shifted = pltpu_roll(xb)
    cmp = (xb != shifted).astype(jnp.float32)
    col = jax.lax.broadcasted_iota(jnp.int32, xb.shape, 1)
    w = jnp.where(col == 0, 0.0, jnp.float32(D_MODEL) - col.astype(jnp.float32))
    wd = jnp.sum(cmp * w, axis=-1)

    e0 = s
    e1 = (s / (s + 1e-6)) / jnp.float32(D_MODEL)
    e2 = nz
    e3 = wd / jnp.float32(D_MODEL)

    out = jnp.where(idx == 0, e0,
          jnp.where(idx == 1, e1,
          jnp.where(idx == 2, e2, e3)))
    o_ref[...] = out[:, None]


def pltpu_roll(xb):
    return jnp.roll(xb, 1, axis=1)


@jax.jit
def kernel(x, W, b):
    B, S, D = x.shape
    n_tok = B * S
    x2 = x.reshape(n_tok, D)
    wt = W.T
    b2 = b.reshape(1, N_EXP)
    g = jax.random.gumbel(jax.random.key(42), (B, S, N_EXP),
                          dtype=jnp.float32).reshape(n_tok, N_EXP)

    grid = (n_tok // TOK_TILE,)
    out = pl.pallas_call(
        _moe_body,
        grid=grid,
        in_specs=[
            pl.BlockSpec((TOK_TILE, D), lambda i: (i, 0)),
            pl.BlockSpec((D, N_EXP), lambda i: (0, 0)),
            pl.BlockSpec((1, N_EXP), lambda i: (0, 0)),
            pl.BlockSpec((TOK_TILE, N_EXP), lambda i: (i, 0)),
        ],
        out_specs=pl.BlockSpec((TOK_TILE, 1), lambda i: (i, 0)),
        out_shape=jax.ShapeDtypeStruct((n_tok, 1), jnp.float32),
    )(x2, wt, b2, g)
    return out.reshape(B, S, 1)


# speedup vs baseline: 6.2763x; 1.0924x over previous
import jax
import jax.numpy as jnp
from jax.experimental import pallas as pl

D_MODEL = 4096
N_EXP = 4
TOK_TILE = 512


def _moe_body(x_ref, wt5_ref, b_ref, g_ref, rv_ref, o_ref):
    xb = x_ref[...]
    r = jnp.dot(xb, wt5_ref[...], preferred_element_type=jnp.float32)
    logits = r[:, :N_EXP]
    s = r[:, N_EXP]
    z = (logits + b_ref[...]) + g_ref[...]
    idx = jnp.argmax(z, axis=-1)

    nzm = (xb != 0.0).astype(jnp.float32)
    cmpm = (xb != jnp.roll(xb, 1, axis=1)).astype(jnp.float32)
    nz = jnp.dot(nzm, rv_ref[...],
                 preferred_element_type=jnp.float32)[:, 0]
    wd = jnp.dot(cmpm, rv_ref[...],
                 preferred_element_type=jnp.float32)[:, 1]

    e0 = s
    e1 = (s / (s + 1e-6)) / jnp.float32(D_MODEL)
    e2 = nz
    e3 = wd / jnp.float32(D_MODEL)

    out = jnp.where(idx == 0, e0,
          jnp.where(idx == 1, e1,
          jnp.where(idx == 2, e2, e3)))
    o_ref[...] = out[:, None]


@jax.jit
def kernel(x, W, b):
    B, S, D = x.shape
    n_tok = B * S
    x2 = x.reshape(n_tok, D)
    wt5 = jnp.concatenate([W.T, jnp.ones((D, 1), jnp.float32)], axis=1)
    b2 = b.reshape(1, N_EXP)
    g = jax.random.gumbel(jax.random.key(42), (B, S, N_EXP),
                          dtype=jnp.float32).reshape(n_tok, N_EXP)
    i = jnp.arange(D, dtype=jnp.float32)
    w = jnp.where(i == 0, 0.0, jnp.float32(D) - i)
    rv = jnp.stack([jnp.ones((D,), jnp.float32), w], axis=1)

    grid = (n_tok // TOK_TILE,)
    out = pl.pallas_call(
        _moe_body,
        grid=grid,
        in_specs=[
            pl.BlockSpec((TOK_TILE, D), lambda i: (i, 0)),
            pl.BlockSpec((D, N_EXP + 1), lambda i: (0, 0)),
            pl.BlockSpec((1, N_EXP), lambda i: (0, 0)),
            pl.BlockSpec((TOK_TILE, N_EXP), lambda i: (i, 0)),
            pl.BlockSpec((D, 2), lambda i: (0, 0)),
        ],
        out_specs=pl.BlockSpec((TOK_TILE, 1), lambda i: (i, 0)),
        out_shape=jax.ShapeDtypeStruct((n_tok, 1), jnp.float32),
    )(x2, wt5, b2, g, rv)
    return out.reshape(B, S, 1)
